# Initial kernel scaffold; baseline (speedup 1.0000x reference)
import functools

import jax
import jax.numpy as jnp
from jax import lax
from jax.experimental import pallas as pl
from jax.experimental.pallas import tpu as pltpu
from jax.experimental.pallas import tpu_sc as plsc

NC = 2
NS = 16
NW = NC * NS
CHUNK = 80
ROWS_PER_TILE = 640


def _sc_mesh():
    return plsc.VectorSubcoreMesh(
        core_axis_name="c", subcore_axis_name="s", num_cores=NC, num_subcores=NS
    )


def _make_deg_kernel(E, NP):
    e_per_w = E // NW
    n_chunks = e_per_w // CHUNK

    @functools.partial(
        pl.kernel,
        out_type=jax.ShapeDtypeStruct((NC, NP), jnp.float32),
        mesh=_sc_mesh(),
        scratch_types=[
            pltpu.VMEM((CHUNK,), jnp.int32),
            pltpu.VMEM((CHUNK,), jnp.float32),
            pltpu.VMEM((ROWS_PER_TILE,), jnp.float32),
            pltpu.MemorySpace.VMEM_SHARED((NP,), jnp.float32),
        ],
    )
    def deg_kernel(dst_hbm, ones_hbm, zeros_hbm, out_hbm, idx_v, ones_v, buf_v, acc):
        cid = lax.axis_index("c")
        sid = lax.axis_index("s")
        wid = cid * NS + sid
        row0 = pl.multiple_of(sid * ROWS_PER_TILE, 8)
        pltpu.sync_copy(zeros_hbm, buf_v)
        pltpu.sync_copy(ones_hbm, ones_v)
        pltpu.sync_copy(buf_v, acc.at[pl.ds(row0, ROWS_PER_TILE)])
        plsc.subcore_barrier()
        base = wid * e_per_w

        def body(c, carry):
            off = pl.multiple_of(base + c * CHUNK, 8)
            pltpu.sync_copy(dst_hbm.at[pl.ds(off, CHUNK)], idx_v)
            pltpu.sync_copy(ones_v, acc.at[idx_v], add=True)
            return carry

        lax.fori_loop(0, n_chunks, body, 0)
        plsc.subcore_barrier()
        pltpu.sync_copy(acc.at[pl.ds(row0, ROWS_PER_TILE)], buf_v)
        pltpu.sync_copy(buf_v, out_hbm.at[cid, pl.ds(row0, ROWS_PER_TILE)])

    return deg_kernel


def _make_agg_kernel(E, NP, D):
    e_per_w = E // NW
    n_chunks = e_per_w // CHUNK

    @functools.partial(
        pl.kernel,
        out_type=jax.ShapeDtypeStruct((NC, NP, D), jnp.float32),
        mesh=_sc_mesh(),
        scratch_types=[
            pltpu.VMEM((CHUNK,), jnp.int32),
            pltpu.VMEM((CHUNK,), jnp.int32),
            pltpu.VMEM((CHUNK, D), jnp.float32),
            pltpu.VMEM((128, D), jnp.float32),
            pltpu.MemorySpace.VMEM_SHARED((NP, D), jnp.float32),
            pltpu.SemaphoreType.DMA,
        ],
    )
    def agg_kernel(x_hbm, src_hbm, dst_hbm, zeros_hbm, out_hbm,
                   isrc, idst, rows, buf_v, acc, sem):
        cid = lax.axis_index("c")
        sid = lax.axis_index("s")
        wid = cid * NS + sid
        row0 = pl.multiple_of(sid * ROWS_PER_TILE, 8)
        pltpu.sync_copy(zeros_hbm, buf_v)
        for j in range(ROWS_PER_TILE // 128):
            pltpu.sync_copy(buf_v, acc.at[pl.ds(row0 + j * 128, 128)])
        plsc.subcore_barrier()
        base = wid * e_per_w

        def body(c, carry):
            off = pl.multiple_of(base + c * CHUNK, 8)
            pltpu.sync_copy(src_hbm.at[pl.ds(off, CHUNK)], isrc)
            pltpu.sync_copy(dst_hbm.at[pl.ds(off, CHUNK)], idst)
            pltpu.async_copy(x_hbm.at[isrc], rows, sem).wait()
            pltpu.sync_copy(rows, acc.at[idst], add=True)
            return carry

        lax.fori_loop(0, n_chunks, body, 0)
        plsc.subcore_barrier()
        for j in range(ROWS_PER_TILE // 128):
            r = pl.multiple_of(row0 + j * 128, 8)
            pltpu.sync_copy(acc.at[pl.ds(r, 128)], buf_v)
            pltpu.sync_copy(buf_v, out_hbm.at[cid, pl.ds(r, 128)])

    return agg_kernel


def _prep_body(deg_ref, x_ref, xs_ref):
    dinv = lax.rsqrt(deg_ref[...])
    xs_ref[...] = x_ref[...] * dinv


def _mm_body(aggp_ref, xs_ref, deg_ref, w1_ref, b1_ref, w2_ref, gs_ref):
    dinv = lax.rsqrt(deg_ref[...])
    a1 = (aggp_ref[0] + aggp_ref[1] + xs_ref[...]) * dinv
    h = jnp.dot(a1, w1_ref[...], preferred_element_type=jnp.float32,
                precision=lax.Precision.HIGHEST)
    h = jnp.maximum(h + b1_ref[...], 0.0)
    g = jnp.dot(h, w2_ref[...], preferred_element_type=jnp.float32,
                precision=lax.Precision.HIGHEST)
    gs_ref[...] = g * dinv


def _final_body(aggp_ref, gs_ref, deg_ref, b2_ref, o_ref):
    dinv = lax.rsqrt(deg_ref[...])
    z = (aggp_ref[0] + aggp_ref[1] + gs_ref[...]) * dinv + b2_ref[...]
    m = jnp.max(z, axis=1, keepdims=True)
    e = jnp.exp(z - m)
    s = jnp.sum(e, axis=1, keepdims=True)
    o_ref[...] = z - m - jnp.log(s)


def kernel(x, edge_index, W1, b1, W2, b2):
    N, DIN = x.shape
    DHID = W1.shape[1]
    DOUT = W2.shape[1]
    E = edge_index.shape[1]
    NP = -(-N // (NW * ROWS_PER_TILE // NS)) * (NW * ROWS_PER_TILE // NS)
    BM = NP // 5

    ei = edge_index.astype(jnp.int32)
    src = ei[0]
    dst = ei[1]
    x_p = jnp.pad(x, ((0, NP - N), (0, 0)))

    degp = _make_deg_kernel(E, NP)(
        dst,
        jnp.ones((CHUNK,), jnp.float32),
        jnp.zeros((ROWS_PER_TILE,), jnp.float32),
    )
    deg_col = (degp[0] + degp[1] + 1.0)[:, None]

    xs = pl.pallas_call(
        _prep_body,
        grid=(NP // BM,),
        in_specs=[
            pl.BlockSpec((BM, 1), lambda m: (m, 0)),
            pl.BlockSpec((BM, DIN), lambda m: (m, 0)),
        ],
        out_specs=pl.BlockSpec((BM, DIN), lambda m: (m, 0)),
        out_shape=jax.ShapeDtypeStruct((NP, DIN), jnp.float32),
    )(deg_col, x_p)

    agg1 = _make_agg_kernel(E, NP, DIN)(
        xs, src, dst, jnp.zeros((128, DIN), jnp.float32)
    )

    gs = pl.pallas_call(
        _mm_body,
        grid=(NP // BM,),
        in_specs=[
            pl.BlockSpec((NC, BM, DIN), lambda m: (0, m, 0)),
            pl.BlockSpec((BM, DIN), lambda m: (m, 0)),
            pl.BlockSpec((BM, 1), lambda m: (m, 0)),
            pl.BlockSpec((DIN, DHID), lambda m: (0, 0)),
            pl.BlockSpec((1, DHID), lambda m: (0, 0)),
            pl.BlockSpec((DHID, DOUT), lambda m: (0, 0)),
        ],
        out_specs=pl.BlockSpec((BM, DOUT), lambda m: (m, 0)),
        out_shape=jax.ShapeDtypeStruct((NP, DOUT), jnp.float32),
    )(agg1, xs, deg_col, W1, b1.reshape(1, DHID), W2)

    agg2 = _make_agg_kernel(E, NP, DOUT)(
        gs, src, dst, jnp.zeros((128, DOUT), jnp.float32)
    )

    out = pl.pallas_call(
        _final_body,
        grid=(NP // BM,),
        in_specs=[
            pl.BlockSpec((NC, BM, DOUT), lambda m: (0, m, 0)),
            pl.BlockSpec((BM, DOUT), lambda m: (m, 0)),
            pl.BlockSpec((BM, 1), lambda m: (m, 0)),
            pl.BlockSpec((1, DOUT), lambda m: (0, 0)),
        ],
        out_specs=pl.BlockSpec((BM, DOUT), lambda m: (0, 0)) if False else pl.BlockSpec((BM, DOUT), lambda m: (m, 0)),
        out_shape=jax.ShapeDtypeStruct((NP, DOUT), jnp.float32),
    )(agg2, gs, deg_col, b2.reshape(1, DOUT))

    return out[:N]


# speedup vs baseline: 14.1624x; 14.1624x over previous
import functools

import jax
import jax.numpy as jnp
from jax import lax
from jax.experimental import pallas as pl
from jax.experimental.pallas import tpu as pltpu
from jax.experimental.pallas import tpu_sc as plsc

NC = 2
NS = 16
NW = NC * NS
CHUNK = 80
ROWS_PER_TILE = 640


def _sc_mesh():
    return plsc.VectorSubcoreMesh(
        core_axis_name="c", subcore_axis_name="s", num_cores=NC, num_subcores=NS
    )


def _make_deg_kernel(E, NP):
    e_per_w = E // NW
    n_chunks = e_per_w // CHUNK

    @functools.partial(
        pl.kernel,
        out_type=jax.ShapeDtypeStruct((NC, NP), jnp.float32),
        mesh=_sc_mesh(),
        scratch_types=[
            pltpu.VMEM((CHUNK,), jnp.int32),
            pltpu.VMEM((CHUNK,), jnp.float32),
            pltpu.VMEM((ROWS_PER_TILE,), jnp.float32),
            pltpu.MemorySpace.VMEM_SHARED((NP,), jnp.float32),
        ],
    )
    def deg_kernel(dst_hbm, ones_hbm, zeros_hbm, out_hbm, idx_v, ones_v, buf_v, acc):
        cid = lax.axis_index("c")
        sid = lax.axis_index("s")
        wid = cid * NS + sid
        row0 = pl.multiple_of(sid * ROWS_PER_TILE, 8)
        pltpu.sync_copy(zeros_hbm, buf_v)
        pltpu.sync_copy(ones_hbm, ones_v)
        pltpu.sync_copy(buf_v, acc.at[pl.ds(row0, ROWS_PER_TILE)])
        plsc.subcore_barrier()
        base = wid * e_per_w

        def body(c, carry):
            off = pl.multiple_of(base + c * CHUNK, 8)
            pltpu.sync_copy(dst_hbm.at[pl.ds(off, CHUNK)], idx_v)
            pltpu.sync_copy(ones_v, acc.at[idx_v], add=True)
            return carry

        lax.fori_loop(0, n_chunks, body, 0)
        plsc.subcore_barrier()
        pltpu.sync_copy(acc.at[pl.ds(row0, ROWS_PER_TILE)], buf_v)
        pltpu.sync_copy(buf_v, out_hbm.at[cid, pl.ds(row0, ROWS_PER_TILE)])

    return deg_kernel


def _make_agg_kernel(E, NP, D):
    e_per_w = E // NW
    n_chunks = e_per_w // CHUNK

    @functools.partial(
        pl.kernel,
        out_type=jax.ShapeDtypeStruct((NC, NP, D), jnp.float32),
        mesh=_sc_mesh(),
        scratch_types=[
            pltpu.VMEM((CHUNK,), jnp.int32),
            pltpu.VMEM((CHUNK,), jnp.int32),
            pltpu.VMEM((CHUNK, D), jnp.float32),
            pltpu.VMEM((128, D), jnp.float32),
            pltpu.MemorySpace.VMEM_SHARED((NP, D), jnp.float32),
            pltpu.SemaphoreType.DMA,
        ],
    )
    def agg_kernel(x_hbm, src_hbm, dst_hbm, zeros_hbm, out_hbm,
                   isrc, idst, rows, buf_v, acc, sem):
        cid = lax.axis_index("c")
        sid = lax.axis_index("s")
        wid = cid * NS + sid
        row0 = pl.multiple_of(sid * ROWS_PER_TILE, 8)
        pltpu.sync_copy(zeros_hbm, buf_v)
        for j in range(ROWS_PER_TILE // 128):
            pltpu.sync_copy(buf_v, acc.at[pl.ds(row0 + j * 128, 128)])
        plsc.subcore_barrier()
        base = wid * e_per_w

        def body(c, carry):
            off = pl.multiple_of(base + c * CHUNK, 8)
            pltpu.sync_copy(src_hbm.at[pl.ds(off, CHUNK)], isrc)
            pltpu.sync_copy(dst_hbm.at[pl.ds(off, CHUNK)], idst)
            pltpu.async_copy(x_hbm.at[isrc], rows, sem).wait()
            pltpu.sync_copy(rows, acc.at[idst], add=True)
            return carry

        lax.fori_loop(0, n_chunks, body, 0)
        plsc.subcore_barrier()
        for j in range(ROWS_PER_TILE // 128):
            r = pl.multiple_of(row0 + j * 128, 8)
            pltpu.sync_copy(acc.at[pl.ds(r, 128)], buf_v)
            pltpu.sync_copy(buf_v, out_hbm.at[cid, pl.ds(r, 128)])

    return agg_kernel


def _prep_body(deg_ref, x_ref, xs_ref):
    dinv = lax.rsqrt(deg_ref[...])
    xs_ref[...] = x_ref[...] * dinv


def _mm_body(aggp_ref, xs_ref, deg_ref, w1_ref, b1_ref, w2_ref, gs_ref):
    dinv = lax.rsqrt(deg_ref[...])
    a1 = (aggp_ref[0] + aggp_ref[1] + xs_ref[...]) * dinv
    h = jnp.dot(a1, w1_ref[...], preferred_element_type=jnp.float32,
                precision=lax.Precision.HIGHEST)
    h = jnp.maximum(h + b1_ref[...], 0.0)
    g = jnp.dot(h, w2_ref[...], preferred_element_type=jnp.float32,
                precision=lax.Precision.HIGHEST)
    gs = g * dinv
    gs_ref[...] = jnp.concatenate(
        [gs, jnp.zeros((gs.shape[0], 128 - gs.shape[1]), jnp.float32)], axis=1)


def _final_body(aggp_ref, gs_ref, deg_ref, b2_ref, o_ref):
    dout = b2_ref.shape[1]
    dinv = lax.rsqrt(deg_ref[...])
    zf = aggp_ref[0] + aggp_ref[1] + gs_ref[...]
    z = zf[:, :dout] * dinv + b2_ref[...]
    m = jnp.max(z, axis=1, keepdims=True)
    e = jnp.exp(z - m)
    s = jnp.sum(e, axis=1, keepdims=True)
    o_ref[...] = z - m - jnp.log(s)


def kernel(x, edge_index, W1, b1, W2, b2):
    N, DIN = x.shape
    DHID = W1.shape[1]
    DOUT = W2.shape[1]
    E = edge_index.shape[1]
    NP = NS * ROWS_PER_TILE
    assert N <= NP and E % (NW * CHUNK) == 0
    BM = NP // 5

    ei = edge_index.astype(jnp.int32)
    src = ei[0]
    dst = ei[1]
    x_p = jnp.pad(x, ((0, NP - N), (0, 0)))

    degp = _make_deg_kernel(E, NP)(
        dst,
        jnp.ones((CHUNK,), jnp.float32),
        jnp.zeros((ROWS_PER_TILE,), jnp.float32),
    )
    deg_col = (degp[0] + degp[1] + 1.0)[:, None]

    xs = pl.pallas_call(
        _prep_body,
        grid=(NP // BM,),
        in_specs=[
            pl.BlockSpec((BM, 1), lambda m: (m, 0)),
            pl.BlockSpec((BM, DIN), lambda m: (m, 0)),
        ],
        out_specs=pl.BlockSpec((BM, DIN), lambda m: (m, 0)),
        out_shape=jax.ShapeDtypeStruct((NP, DIN), jnp.float32),
    )(deg_col, x_p)

    agg1 = _make_agg_kernel(E, NP, DIN)(
        xs, src, dst, jnp.zeros((128, DIN), jnp.float32)
    )

    gs = pl.pallas_call(
        _mm_body,
        grid=(NP // BM,),
        in_specs=[
            pl.BlockSpec((NC, BM, DIN), lambda m: (0, m, 0)),
            pl.BlockSpec((BM, DIN), lambda m: (m, 0)),
            pl.BlockSpec((BM, 1), lambda m: (m, 0)),
            pl.BlockSpec((DIN, DHID), lambda m: (0, 0)),
            pl.BlockSpec((1, DHID), lambda m: (0, 0)),
            pl.BlockSpec((DHID, DOUT), lambda m: (0, 0)),
        ],
        out_specs=pl.BlockSpec((BM, 128), lambda m: (m, 0)),
        out_shape=jax.ShapeDtypeStruct((NP, 128), jnp.float32),
    )(agg1, xs, deg_col, W1, b1.reshape(1, DHID), W2)

    agg2 = _make_agg_kernel(E, NP, 128)(
        gs, src, dst, jnp.zeros((128, 128), jnp.float32)
    )

    out = pl.pallas_call(
        _final_body,
        grid=(NP // BM,),
        in_specs=[
            pl.BlockSpec((NC, BM, 128), lambda m: (0, m, 0)),
            pl.BlockSpec((BM, 128), lambda m: (m, 0)),
            pl.BlockSpec((BM, 1), lambda m: (m, 0)),
            pl.BlockSpec((1, DOUT), lambda m: (0, 0)),
        ],
        out_specs=pl.BlockSpec((BM, DOUT), lambda m: (m, 0)),
        out_shape=jax.ShapeDtypeStruct((NP, DOUT), jnp.float32),
    )(agg2, gs, deg_col, b2.reshape(1, DOUT))

    return out[:N]
